# baseline (speedup 1.0000x reference)
import functools

import jax
import jax.numpy as jnp
from jax import lax
from jax.experimental import pallas as pl
from jax.experimental.pallas import tpu as pltpu
from jax.experimental.pallas import tpu_sc as plsc

_N_PAIRS = 98304
_DIM = 128
_LANES = 16

_info = plsc.get_sparse_core_info()
_NC = _info.num_cores
_NS = _info.num_subcores
_NW = _NC * _NS
_PER_W = _N_PAIRS // _NW
_CHUNK = 128
_NCH = _PER_W // _CHUNK
_PGRP = _LANES * (_LANES + 1)


def _sc_pair_dots(src3, tgt3, nodes_embed, context_nodes_embed):
  mesh = plsc.VectorSubcoreMesh(core_axis_name="c", subcore_axis_name="s")

  @functools.partial(
      pl.kernel,
      mesh=mesh,
      compiler_params=pltpu.CompilerParams(needs_layout_passes=False),
      out_type=jax.ShapeDtypeStruct((_NW, _NCH, _CHUNK), jnp.float32),
      scratch_types=[
          pltpu.VMEM((_NCH, _CHUNK), jnp.int32),
          pltpu.VMEM((_NCH, _CHUNK), jnp.int32),
          pltpu.VMEM((_CHUNK, _DIM), jnp.float32),
          pltpu.VMEM((_CHUNK, _DIM), jnp.float32),
          pltpu.VMEM((_CHUNK, _DIM), jnp.float32),
          pltpu.VMEM((_CHUNK, _DIM), jnp.float32),
          pltpu.VMEM((_NCH, _CHUNK), jnp.float32),
          pltpu.VMEM((_CHUNK // _LANES * _PGRP,), jnp.float32),
          pltpu.SemaphoreType.DMA,
          pltpu.SemaphoreType.DMA,
          pltpu.SemaphoreType.DMA,
          pltpu.SemaphoreType.DMA,
      ],
  )
  def dots(src_hbm, tgt_hbm, nodes_hbm, ctx_hbm, out_hbm,
           sidx, tidx, srows0, srows1, trows0, trows1,
           outbuf, pscr, sem_s0, sem_s1, sem_t0, sem_t1):
    wid = lax.axis_index("s") * _NC + lax.axis_index("c")
    pltpu.sync_copy(src_hbm.at[wid], sidx)
    pltpu.sync_copy(tgt_hbm.at[wid], tidx)
    lanes = lax.iota(jnp.int32, _LANES)
    srows = (srows0, srows1)
    trows = (trows0, trows1)
    sem_s = (sem_s0, sem_s1)
    sem_t = (sem_t0, sem_t1)

    def start(j, b):
      pltpu.async_copy(nodes_hbm.at[sidx.at[j]], srows[b], sem_s[b])
      pltpu.async_copy(ctx_hbm.at[tidx.at[j]], trows[b], sem_t[b])

    def wait(b):
      pltpu.make_async_copy(nodes_hbm.at[sidx.at[0]], srows[b], sem_s[b]).wait()
      pltpu.make_async_copy(ctx_hbm.at[tidx.at[0]], trows[b], sem_t[b]).wait()

    def compute(j, b):
      @plsc.parallel_loop(0, _CHUNK // _LANES, unroll=4)
      def group_body(g):
        base = g * _PGRP
        for r in range(_LANES):
          row = g * _LANES + r
          p = srows[b][row, pl.ds(0, _LANES)] * trows[b][row, pl.ds(0, _LANES)]
          for k in range(1, _DIM // _LANES):
            p = p + (srows[b][row, pl.ds(k * _LANES, _LANES)]
                     * trows[b][row, pl.ds(k * _LANES, _LANES)])
          pscr[pl.ds(base + r * (_LANES + 1), _LANES)] = p
        stride = base + lanes * (_LANES + 1)
        q = plsc.load_gather(pscr, [stride])
        for d in range(1, _LANES):
          q = q + plsc.load_gather(pscr, [stride + d])
        outbuf[j, pl.ds(g * _LANES, _LANES)] = q

    start(0, 0)

    def chunk_pair(j2, carry):
      j = j2 * 2
      start(j + 1, 1)
      wait(0)
      compute(j, 0)

      @pl.when(j + 2 < _NCH)
      def _():
        start(j + 2, 0)

      wait(1)
      compute(j + 1, 1)
      return carry

    lax.fori_loop(0, _NCH // 2, chunk_pair, 0)
    pltpu.sync_copy(outbuf, out_hbm.at[wid])

  return dots(src3, tgt3, nodes_embed, context_nodes_embed)


def _tc_loss(inner, label):
  rows = _N_PAIRS // 128
  x2 = inner.reshape(rows, 128)
  l2 = label.reshape(rows, 128)

  def body(x_ref, l_ref, o_ref):
    z = x_ref[...] * l_ref[...]
    ls = jnp.minimum(z, 0.0) - jnp.log(1.0 + jnp.exp(-jnp.abs(z)))
    o_ref[0, 0] = -jnp.sum(ls) / _N_PAIRS

  out = pl.pallas_call(
      body,
      out_shape=jax.ShapeDtypeStruct((1, 1), jnp.float32),
      out_specs=pl.BlockSpec(memory_space=pltpu.SMEM),
  )(x2, l2)
  return out[0, 0]


def kernel(source_node, target_node, label, nodes_embed, context_nodes_embed):
  src3 = source_node.astype(jnp.int32).reshape(_NW, _NCH, _CHUNK)
  tgt3 = target_node.astype(jnp.int32).reshape(_NW, _NCH, _CHUNK)
  inner3 = _sc_pair_dots(src3, tgt3, nodes_embed, context_nodes_embed)
  return _tc_loss(inner3.reshape(_N_PAIRS), label)


# speedup vs baseline: 1.3299x; 1.3299x over previous
import functools

import jax
import jax.numpy as jnp
from jax import lax
from jax.experimental import pallas as pl
from jax.experimental.pallas import tpu as pltpu
from jax.experimental.pallas import tpu_sc as plsc

_N_PAIRS = 98304
_DIM = 128
_LANES = 16

_info = plsc.get_sparse_core_info()
_NC = _info.num_cores
_NS = _info.num_subcores
_NW = _NC * _NS
_PER_W = _N_PAIRS // _NW
_CHUNK = 128
_NCH = _PER_W // _CHUNK
_PGRP = _LANES * (_LANES + 1)


def _sc_pair_dots(src3, tgt3, nodes_embed, context_nodes_embed):
  mesh = plsc.VectorSubcoreMesh(core_axis_name="c", subcore_axis_name="s")

  @functools.partial(
      pl.kernel,
      mesh=mesh,
      compiler_params=pltpu.CompilerParams(needs_layout_passes=False),
      out_type=jax.ShapeDtypeStruct((_NW, _NCH, _CHUNK), jnp.float32),
      scratch_types=[
          pltpu.VMEM((_NCH, _CHUNK), jnp.int32),
          pltpu.VMEM((_NCH, _CHUNK), jnp.int32),
          pltpu.VMEM((_CHUNK, _DIM), jnp.float32),
          pltpu.VMEM((_CHUNK, _DIM), jnp.float32),
          pltpu.VMEM((_CHUNK, _DIM), jnp.float32),
          pltpu.VMEM((_CHUNK, _DIM), jnp.float32),
          pltpu.VMEM((_NCH, _CHUNK), jnp.float32),
          pltpu.VMEM((_CHUNK // _LANES * _PGRP,), jnp.float32),
          pltpu.SemaphoreType.DMA,
          pltpu.SemaphoreType.DMA,
          pltpu.SemaphoreType.DMA,
          pltpu.SemaphoreType.DMA,
      ],
  )
  def dots(src_hbm, tgt_hbm, nodes_hbm, ctx_hbm, out_hbm,
           sidx, tidx, srows0, srows1, trows0, trows1,
           outbuf, pscr, sem_s0, sem_s1, sem_t0, sem_t1):
    wid = lax.axis_index("s") * _NC + lax.axis_index("c")
    pltpu.sync_copy(src_hbm.at[wid], sidx)
    pltpu.sync_copy(tgt_hbm.at[wid], tidx)
    lanes = lax.iota(jnp.int32, _LANES)
    srows = (srows0, srows1)
    trows = (trows0, trows1)
    sem_s = (sem_s0, sem_s1)
    sem_t = (sem_t0, sem_t1)

    def start(j, b):
      pltpu.async_copy(nodes_hbm.at[sidx.at[j]], srows[b], sem_s[b])
      pltpu.async_copy(ctx_hbm.at[tidx.at[j]], trows[b], sem_t[b])

    def wait(b):
      pltpu.make_async_copy(nodes_hbm.at[sidx.at[0]], srows[b], sem_s[b]).wait()
      pltpu.make_async_copy(ctx_hbm.at[tidx.at[0]], trows[b], sem_t[b]).wait()

    def compute(j, b):
      @plsc.parallel_loop(0, _CHUNK // _LANES, unroll=2)
      def group_body(g):
        base = g * _PGRP
        for r in range(_LANES):
          row = g * _LANES + r
          p = srows[b][row, pl.ds(0, _LANES)] * trows[b][row, pl.ds(0, _LANES)]
          for k in range(1, _DIM // _LANES):
            p = p + (srows[b][row, pl.ds(k * _LANES, _LANES)]
                     * trows[b][row, pl.ds(k * _LANES, _LANES)])
          pscr[pl.ds(base + r * (_LANES + 1), _LANES)] = p
        stride = base + lanes * (_LANES + 1)
        q = plsc.load_gather(pscr, [stride])
        for d in range(1, _LANES):
          q = q + plsc.load_gather(pscr, [stride + d])
        outbuf[j, pl.ds(g * _LANES, _LANES)] = q

    start(0, 0)

    def chunk_pair(j2, carry):
      j = j2 * 2
      start(j + 1, 1)
      wait(0)
      compute(j, 0)

      @pl.when(j + 2 < _NCH)
      def _():
        start(j + 2, 0)

      wait(1)
      compute(j + 1, 1)
      return carry

    lax.fori_loop(0, _NCH // 2, chunk_pair, 0)
    pltpu.sync_copy(outbuf, out_hbm.at[wid])

  return dots(src3, tgt3, nodes_embed, context_nodes_embed)


def _tc_loss(inner, label):
  rows = _N_PAIRS // 128
  x2 = inner.reshape(rows, 128)
  l2 = label.reshape(rows, 128)

  def body(x_ref, l_ref, o_ref):
    z = x_ref[...] * l_ref[...]
    ls = jnp.minimum(z, 0.0) - jnp.log(1.0 + jnp.exp(-jnp.abs(z)))
    o_ref[0, 0] = -jnp.sum(ls) / _N_PAIRS

  out = pl.pallas_call(
      body,
      out_shape=jax.ShapeDtypeStruct((1, 1), jnp.float32),
      out_specs=pl.BlockSpec(memory_space=pltpu.SMEM),
  )(x2, l2)
  return out[0, 0]


def kernel(source_node, target_node, label, nodes_embed, context_nodes_embed):
  src3 = source_node.astype(jnp.int32).reshape(_NW, _NCH, _CHUNK)
  tgt3 = target_node.astype(jnp.int32).reshape(_NW, _NCH, _CHUNK)
  inner3 = _sc_pair_dots(src3, tgt3, nodes_embed, context_nodes_embed)
  return _tc_loss(inner3.reshape(_N_PAIRS), label)
